# baseline (speedup 1.0000x reference)
import functools

import jax
import jax.numpy as jnp
from jax import lax
from jax.experimental import pallas as pl
from jax.experimental.pallas import tpu as pltpu
from jax.experimental.pallas import tpu_sc as plsc

V = 10000
C = 40000
E = 160000
FM = 128
QM = 128
ROUNDS = 4
EP = 163840
CHUNKS = EP // 16 // 128
BV = 1000
BC = 2000
NBV = V // BV
NBC = C // BC
S_LOSS = 1.0 / (C * QM)


def _mm(x, w):
    return lax.dot_general(x, w, (((1,), (0,)), ((), ())),
                           preferred_element_type=jnp.float32)


def _softplus(x):
    return jnp.maximum(x, 0.0) + jnp.log1p(jnp.exp(-jnp.abs(x)))


QR = 10000
RPT_HI = 632
RPT_TAIL = QR - 15 * RPT_HI


def _make_segsum(n_quarters):
    n_per_core = n_quarters // 2
    mesh = plsc.VectorSubcoreMesh(core_axis_name="c", subcore_axis_name="s")
    n_out = n_quarters * QR

    @functools.partial(
        pl.kernel,
        mesh=mesh,
        out_type=jax.ShapeDtypeStruct((n_out, 128), jnp.float32),
        scratch_types=[
            pltpu.VMEM((CHUNKS, 128), jnp.int32),
            pltpu.VMEM((CHUNKS, 128), jnp.int32),
            pltpu.VMEM((128, 128), jnp.float32),
            pltpu.VMEM((64, 128), jnp.float32),
            pltpu.VMEM_SHARED((QR + 8, 128), jnp.float32),
            pltpu.SemaphoreType.DMA,
        ],
    )
    def segsum(table, gidx, zsrc, *rest):
        sqs, (out, gv, sv, buf, zbuf, acc, sem) = rest[:n_quarters], rest[n_quarters:]
        c = lax.axis_index("c")
        s = lax.axis_index("s")
        pltpu.sync_copy(gidx.at[pl.ds(s * CHUNKS, CHUNKS)], gv)
        pltpu.sync_copy(zsrc, zbuf)

        def run_pass(sq, q):
            pltpu.sync_copy(sq.at[pl.ds(s * CHUNKS, CHUNKS)], sv)
            base = s * RPT_HI

            def zero_rows(nrows):
                off = 0
                while off < nrows:
                    n = min(64, nrows - off)
                    pltpu.sync_copy(zbuf.at[pl.ds(0, n)],
                                    acc.at[pl.ds(base + off, n)])
                    off += n

            @pl.when(s < 15)
            def _():
                zero_rows(RPT_HI)

            @pl.when(s == 15)
            def _():
                zero_rows(RPT_TAIL)
                pltpu.sync_copy(zbuf.at[pl.ds(0, 8)], acc.at[pl.ds(QR, 8)])

            plsc.subcore_barrier()

            def step(j, carry):
                pltpu.async_copy(table.at[gv.at[j]], buf, sem).wait()
                pltpu.sync_copy(buf, acc.at[sv.at[j]], add=True)
                return carry

            lax.fori_loop(0, CHUNKS, step, 0)
            plsc.subcore_barrier()
            obase = q * QR + base

            @pl.when(s < 15)
            def _():
                pltpu.sync_copy(acc.at[pl.ds(base, RPT_HI)],
                                out.at[pl.ds(obase, RPT_HI)])

            @pl.when(s == 15)
            def _():
                pltpu.sync_copy(acc.at[pl.ds(base, RPT_TAIL)],
                                out.at[pl.ds(obase, RPT_TAIL)])

        for cid in range(2):
            @pl.when(c == cid)
            def _(cid=cid):
                for p in range(n_per_core):
                    run_pass(sqs[cid * n_per_core + p], cid * n_per_core + p)

    def call(table, gidx, sqs, zsrc):
        return segsum(table, gidx, zsrc, *sqs)

    return call


_segsum_c = _make_segsum(4)
_segsum_l = _make_segsum(2)


def _k1_body(var_ref, nv_ref, w1v, w1n, b1, w2, b2, vq_ref):
    h = _mm(var_ref[...], w1v[...]) + _mm(nv_ref[...], w1n[...]) + b1[...]
    h = jnp.maximum(h, 0.0)
    vq_ref[...] = _mm(h, w2[...]) + b2[...]


def _k1b_body(vq_ref, lits_ref):
    sign = jnp.where(pl.program_id(0) == 0, 1.0, -1.0)
    lits_ref[...] = _softplus(vq_ref[...] * sign)


def _k3_body(cl_ref, nc_ref, cs_ref,
             qw1v, qw1n, qb1, qw2, qb2,
             cw1a, cw1b, cw1c, cb1, cw2, cb2,
             vla_ref, g_ref, y_ref, ps_ref):
    clauses = cl_ref[...]
    h = _mm(clauses, qw1v[...]) + _mm(nc_ref[...], qw1n[...]) + qb1[...]
    h = jnp.maximum(h, 0.0)
    cq = _mm(h, qw2[...]) + qb2[...]
    sig = jax.nn.sigmoid(cq)
    cl = jnp.exp(-cs_ref[...]) * sig
    cgrad = S_LOSS * cl * (1.0 - sig)
    g_ref[...] = -S_LOSS * cl
    u = _mm(clauses, cw1a[...]) + _mm(4.0 * cl, cw1b[...]) \
        + _mm(cgrad, cw1c[...]) + cb1[...]
    u = jnp.maximum(u, 0.0)
    d = _mm(u, cw2[...]) + cb2[...]
    vla_ref[...] = d[:, 0:128]
    y = d[:, 128:256]
    y_ref[...] = y
    ps_ref[...] = jnp.sum(y, axis=0, keepdims=True)[None]


def _k4_body(y_ref, ps_ref, cl_ref, ow1, ob1, ow2, ob2,
             newc_ref, sig_ref, sp_ref):
    mean = jnp.sum(ps_ref[...][:, 0, :], axis=0, keepdims=True) * (1.0 / C)
    yc = y_ref[...] - mean
    var = jnp.mean(yc * yc, axis=1, keepdims=True)
    newc = yc * lax.rsqrt(var + 1e-6) * 0.25 + 0.1 * cl_ref[...]
    newc_ref[...] = newc
    hh = jnp.maximum(_mm(newc, ow1[...]) + ob1[...], 0.0)
    logit = _mm(hh, ow2[...]) + ob2[...]
    sig_ref[...] = jax.nn.sigmoid(logit)
    sp_ref[...] = _softplus(logit)


def _k6_body(glp_ref, gln_ref, vq_ref, var_ref,
             vlp_ref, vln_ref, degp, degn,
             u1a, u1b, u1c, u1d, c1, u2, c2, u3, c3,
             y_ref, ps_ref):
    dpos = degp[...][:, 0:1]
    dneg = degn[...][:, 0:1]
    vdw = 4.0 * lax.rsqrt(jnp.maximum(dpos + dneg, 1.0))
    dwp = lax.rsqrt(jnp.maximum(dpos, 1.0))
    dwn = lax.rsqrt(jnp.maximum(dneg, 1.0))
    vq = vq_ref[...]
    vgrad = (glp_ref[...] * jax.nn.sigmoid(vq)
             - gln_ref[...] * jax.nn.sigmoid(-vq)) * vdw
    vlp = vlp_ref[...] * dwp
    vln = vln_ref[...] * dwn
    u = _mm(vgrad, u1a[...]) + _mm(var_ref[...], u1b[...]) \
        + _mm(vlp, u1c[...]) + _mm(vln, u1d[...]) + c1[...]
    u = jnp.maximum(u, 0.0)
    h2 = jnp.maximum(_mm(u, u2[...]) + c2[...], 0.0)
    y = _mm(h2, u3[...]) + c3[...]
    y_ref[...] = y
    ps_ref[...] = jnp.sum(y, axis=0, keepdims=True)[None]


def _k7_body(y_ref, ps_ref, var_ref, newv_ref):
    mean = jnp.sum(ps_ref[...][:, 0, :], axis=0, keepdims=True) * (1.0 / V)
    yc = y_ref[...] - mean
    var = jnp.mean(yc * yc, axis=1, keepdims=True)
    newv_ref[...] = yc * lax.rsqrt(var + 1e-6) * 0.25 + 0.1 * var_ref[...]


def _row_spec(b, d):
    return pl.BlockSpec((b, d), lambda i: (i, 0))


def _full_spec(shape):
    nd = len(shape)
    return pl.BlockSpec(shape, lambda i, _nd=nd: (0,) * _nd)


def kernel(lit_idx, clause_idx, clauses_graph, variables_graph, params):
    f32 = jnp.float32
    p = params

    qv_w1, qv_b1, qv_w2, qv_b2 = p["variables_query"]
    qc_w1, qc_b1, qc_w2, qc_b2 = p["clauses_query"]
    cm_w1, cm_b1, cm_w2, cm_b2 = p["clause_mlp"]
    ug_w1, ug_b1, ug_w2, ug_b2, ug_w3, ug_b3 = p["update_gate"]
    co_w1, co_b1, co_w2, co_b2 = p["clauses_output"]

    def r2(b):
        return b.reshape(1, -1)

    qv_w1v, qv_w1n = qv_w1[:FM], qv_w1[FM:]
    qc_w1v, qc_w1n = qc_w1[:FM], qc_w1[FM:]
    cm_w1a, cm_w1b, cm_w1c = cm_w1[:FM], cm_w1[FM:FM + QM], cm_w1[FM + QM:]
    ug_w1a, ug_w1b = ug_w1[:QM], ug_w1[QM:QM + FM]
    ug_w1c, ug_w1d = ug_w1[QM + FM:QM + 2 * FM], ug_w1[QM + 2 * FM:]

    pad = EP - E
    lit_g = jnp.pad(lit_idx, (0, pad)).reshape(EP // 128, 128)
    cla_g = jnp.pad(clause_idx, (0, pad)).reshape(EP // 128, 128)
    cla_pad = jnp.pad(clause_idx, (0, pad), constant_values=C)
    lit_pad = jnp.pad(lit_idx, (0, pad), constant_values=2 * V)
    zeros_e = jnp.zeros((EP // 128, 128), jnp.int32)

    def localize(idx, q):
        lo = q * QR
        return jnp.where((idx >= lo) & (idx < lo + QR), idx - lo,
                         QR).reshape(EP // 128, 128).astype(jnp.int32)

    sq_c = [localize(cla_pad, q) for q in range(4)]
    sq_l = [localize(lit_pad, q) for q in range(2)]
    z128 = jnp.zeros((64, 128), f32)
    ones_t = jnp.ones((8, 128), f32)

    deg = _segsum_l(ones_t, zeros_e, sq_l, z128)

    variables = jnp.ones((V, FM), f32)
    clauses = jnp.ones((C, FM), f32)
    base_key = jax.random.key(1)

    k1 = pl.pallas_call(
        _k1_body,
        grid=(NBV,),
        in_specs=[_row_spec(BV, FM), _row_spec(BV, 4),
                  _full_spec((FM, QM)), _full_spec((4, QM)),
                  _full_spec((1, QM)), _full_spec((QM, QM)),
                  _full_spec((1, QM))],
        out_specs=_row_spec(BV, QM),
        out_shape=jax.ShapeDtypeStruct((V, QM), f32),
    )
    k1b = pl.pallas_call(
        _k1b_body,
        grid=(2, NBV),
        in_specs=[pl.BlockSpec((BV, QM), lambda h, i: (i, 0))],
        out_specs=pl.BlockSpec((BV, QM), lambda h, i: (h * NBV + i, 0)),
        out_shape=jax.ShapeDtypeStruct((2 * V, QM), f32),
    )
    k3 = pl.pallas_call(
        _k3_body,
        grid=(NBC,),
        in_specs=[_row_spec(BC, FM), _row_spec(BC, 4), _row_spec(BC, FM)]
        + [_full_spec((FM, QM)), _full_spec((4, QM)), _full_spec((1, QM)),
           _full_spec((QM, QM)), _full_spec((1, QM)),
           _full_spec((FM, 2 * FM)), _full_spec((QM, 2 * FM)),
           _full_spec((QM, 2 * FM)), _full_spec((1, 2 * FM)),
           _full_spec((2 * FM, FM + QM)), _full_spec((1, FM + QM))],
        out_specs=[_row_spec(BC, FM), _row_spec(BC, FM),
                   _row_spec(BC, FM),
                   pl.BlockSpec((1, 1, FM), lambda i: (i, 0, 0))],
        out_shape=[jax.ShapeDtypeStruct((C, FM), f32),
                   jax.ShapeDtypeStruct((C, FM), f32),
                   jax.ShapeDtypeStruct((C, FM), f32),
                   jax.ShapeDtypeStruct((NBC, 1, FM), f32)],
    )
    k4 = pl.pallas_call(
        _k4_body,
        grid=(NBC,),
        in_specs=[_row_spec(BC, FM), _full_spec((NBC, 1, FM)), _row_spec(BC, FM),
                  _full_spec((FM, FM)), _full_spec((1, FM)),
                  _full_spec((FM, 1)), _full_spec((1, 1))],
        out_specs=[_row_spec(BC, FM), _row_spec(BC, 1), _row_spec(BC, 1)],
        out_shape=[jax.ShapeDtypeStruct((C, FM), f32),
                   jax.ShapeDtypeStruct((C, 1), f32),
                   jax.ShapeDtypeStruct((C, 1), f32)],
    )
    neg_spec = pl.BlockSpec((BV, FM), lambda i: (i + NBV, 0))
    k6 = pl.pallas_call(
        _k6_body,
        grid=(NBV,),
        in_specs=[_row_spec(BV, FM), neg_spec,
                  _row_spec(BV, QM), _row_spec(BV, FM),
                  _row_spec(BV, FM), neg_spec,
                  _row_spec(BV, FM), neg_spec,
                  _full_spec((QM, 2 * FM)), _full_spec((FM, 2 * FM)),
                  _full_spec((FM, 2 * FM)), _full_spec((FM, 2 * FM)),
                  _full_spec((1, 2 * FM)),
                  _full_spec((2 * FM, 2 * FM)), _full_spec((1, 2 * FM)),
                  _full_spec((2 * FM, FM)), _full_spec((1, FM))],
        out_specs=[_row_spec(BV, FM),
                   pl.BlockSpec((1, 1, FM), lambda i: (i, 0, 0))],
        out_shape=[jax.ShapeDtypeStruct((V, FM), f32),
                   jax.ShapeDtypeStruct((NBV, 1, FM), f32)],
    )
    k7 = pl.pallas_call(
        _k7_body,
        grid=(NBV,),
        in_specs=[_row_spec(BV, FM), _full_spec((NBV, 1, FM)), _row_spec(BV, FM)],
        out_specs=_row_spec(BV, FM),
        out_shape=jax.ShapeDtypeStruct((V, FM), f32),
    )

    sig_rows = []
    sp_rows = []
    for step in range(ROUNDS):
        nv = jax.random.normal(jax.random.fold_in(base_key, 2 * step), (V, 4), f32)
        nc = jax.random.normal(jax.random.fold_in(base_key, 2 * step + 1), (C, 4), f32)

        vq = k1(variables, nv, qv_w1v, qv_w1n, r2(qv_b1), qv_w2, r2(qv_b2))
        lits = k1b(vq)
        csum = _segsum_c(lits, lit_g, sq_c, z128)
        vla, gcs, y_c, ps_c = k3(
            clauses, nc, csum,
            qc_w1v, qc_w1n, r2(qc_b1), qc_w2, r2(qc_b2),
            cm_w1a, cm_w1b, cm_w1c, r2(cm_b1), cm_w2, r2(cm_b2))
        clauses, sig_l, sp_l = k4(y_c, ps_c, clauses,
                                  co_w1, r2(co_b1), co_w2, r2(co_b2))
        sig_rows.append(sig_l)
        sp_rows.append(sp_l)
        gl = _segsum_l(gcs, cla_g, sq_l, z128)
        vl = _segsum_l(vla, cla_g, sq_l, z128)
        y_v, ps_v = k6(gl, gl, vq, variables, vl, vl, deg, deg,
                       ug_w1a, ug_w1b, ug_w1c, ug_w1d, r2(ug_b1),
                       ug_w2, r2(ug_b2), ug_w3, r2(ug_b3))
        variables = k7(y_v, ps_v, variables)

    sig = jnp.stack([jnp.squeeze(x, -1) for x in sig_rows], axis=0)
    sp = jnp.stack([jnp.squeeze(x, -1) for x in sp_rows], axis=0)
    return sig, sp


# speedup vs baseline: 2.8481x; 2.8481x over previous
import functools

import jax
import jax.numpy as jnp
from jax import lax
from jax.experimental import pallas as pl
from jax.experimental.pallas import tpu as pltpu
from jax.experimental.pallas import tpu_sc as plsc

V = 10000
C = 40000
E = 160000
FM = 128
QM = 128
ROUNDS = 4
EP = 163840
CHUNKS = EP // 16 // 128
BV = 1000
BC = 2000
NBV = V // BV
NBC = C // BC
S_LOSS = 1.0 / (C * QM)


def _mm(x, w):
    return lax.dot_general(x, w, (((1,), (0,)), ((), ())),
                           preferred_element_type=jnp.float32)


def _softplus(x):
    return jnp.maximum(x, 0.0) + jnp.log1p(jnp.exp(-jnp.abs(x)))


QR = 10000
RPT_HI = 632
RPT_TAIL = QR - 15 * RPT_HI


def _make_segsum(n_quarters, gather=True):
    n_per_core = n_quarters // 2
    mesh = plsc.VectorSubcoreMesh(core_axis_name="c", subcore_axis_name="s")
    n_out = n_quarters * QR

    @functools.partial(
        pl.kernel,
        mesh=mesh,
        out_type=jax.ShapeDtypeStruct((n_out, 128), jnp.float32),
        scratch_types=[
            pltpu.VMEM((CHUNKS, 128), jnp.int32),
            pltpu.VMEM((CHUNKS, 128), jnp.int32),
            pltpu.VMEM((128, 128), jnp.float32),
            pltpu.VMEM((64, 128), jnp.float32),
            pltpu.VMEM_SHARED((QR + 8, 128), jnp.float32),
            pltpu.SemaphoreType.DMA,
        ],
    )
    def segsum(table, gidx, zsrc, *rest):
        sqs, (out, gv, sv, buf, zbuf, acc, sem) = rest[:n_quarters], rest[n_quarters:]
        c = lax.axis_index("c")
        s = lax.axis_index("s")
        if gather:
            pltpu.sync_copy(gidx.at[pl.ds(s * CHUNKS, CHUNKS)], gv)
        else:
            pltpu.sync_copy(table, buf)
        pltpu.sync_copy(zsrc, zbuf)

        def run_pass(sq, q):
            pltpu.sync_copy(sq.at[pl.ds(s * CHUNKS, CHUNKS)], sv)
            base = s * RPT_HI

            def zero_rows(nrows):
                off = 0
                while off < nrows:
                    n = min(64, nrows - off)
                    pltpu.sync_copy(zbuf.at[pl.ds(0, n)],
                                    acc.at[pl.ds(base + off, n)])
                    off += n

            @pl.when(s < 15)
            def _():
                zero_rows(RPT_HI)

            @pl.when(s == 15)
            def _():
                zero_rows(RPT_TAIL)
                pltpu.sync_copy(zbuf.at[pl.ds(0, 8)], acc.at[pl.ds(QR, 8)])

            plsc.subcore_barrier()

            def step(j, carry):
                if gather:
                    pltpu.async_copy(table.at[gv.at[j]], buf, sem).wait()
                pltpu.sync_copy(buf, acc.at[sv.at[j]], add=True)
                return carry

            lax.fori_loop(0, CHUNKS, step, 0)
            plsc.subcore_barrier()
            obase = q * QR + base

            @pl.when(s < 15)
            def _():
                pltpu.sync_copy(acc.at[pl.ds(base, RPT_HI)],
                                out.at[pl.ds(obase, RPT_HI)])

            @pl.when(s == 15)
            def _():
                pltpu.sync_copy(acc.at[pl.ds(base, RPT_TAIL)],
                                out.at[pl.ds(obase, RPT_TAIL)])

        for cid in range(2):
            @pl.when(c == cid)
            def _(cid=cid):
                for p in range(n_per_core):
                    run_pass(sqs[cid * n_per_core + p], cid * n_per_core + p)

    def call(table, gidx, sqs, zsrc):
        return segsum(table, gidx, zsrc, *sqs)

    return call


_segsum_c = _make_segsum(4)
_segsum_l = _make_segsum(2)
_segsum_deg = _make_segsum(2, gather=False)


def _k1_body(var_ref, nv_ref, w1v, w1n, b1, w2, b2, vq_ref):
    h = _mm(var_ref[...], w1v[...]) + _mm(nv_ref[...], w1n[...]) + b1[...]
    h = jnp.maximum(h, 0.0)
    vq_ref[...] = _mm(h, w2[...]) + b2[...]


def _k1b_body(vq_ref, lits_ref):
    sign = jnp.where(pl.program_id(0) == 0, 1.0, -1.0)
    lits_ref[...] = _softplus(vq_ref[...] * sign)


def _k3_body(cl_ref, nc_ref, cs_ref,
             qw1v, qw1n, qb1, qw2, qb2,
             cw1a, cw1b, cw1c, cb1, cw2, cb2,
             vla_ref, g_ref, y_ref, ps_ref):
    clauses = cl_ref[...]
    h = _mm(clauses, qw1v[...]) + _mm(nc_ref[...], qw1n[...]) + qb1[...]
    h = jnp.maximum(h, 0.0)
    cq = _mm(h, qw2[...]) + qb2[...]
    sig = jax.nn.sigmoid(cq)
    cl = jnp.exp(-cs_ref[...]) * sig
    cgrad = S_LOSS * cl * (1.0 - sig)
    g_ref[...] = -S_LOSS * cl
    u = _mm(clauses, cw1a[...]) + _mm(4.0 * cl, cw1b[...]) \
        + _mm(cgrad, cw1c[...]) + cb1[...]
    u = jnp.maximum(u, 0.0)
    d = _mm(u, cw2[...]) + cb2[...]
    vla_ref[...] = d[:, 0:128]
    y = d[:, 128:256]
    y_ref[...] = y
    ps_ref[...] = jnp.sum(y, axis=0, keepdims=True)[None]


def _k4_body(y_ref, ps_ref, cl_ref, ow1, ob1, ow2, ob2,
             newc_ref, sig_ref, sp_ref):
    mean = jnp.sum(ps_ref[...][:, 0, :], axis=0, keepdims=True) * (1.0 / C)
    yc = y_ref[...] - mean
    var = jnp.mean(yc * yc, axis=1, keepdims=True)
    newc = yc * lax.rsqrt(var + 1e-6) * 0.25 + 0.1 * cl_ref[...]
    newc_ref[...] = newc
    hh = jnp.maximum(_mm(newc, ow1[...]) + ob1[...], 0.0)
    logit = _mm(hh, ow2[...]) + ob2[...]
    sig_ref[...] = jax.nn.sigmoid(logit)
    sp_ref[...] = _softplus(logit)


def _k6_body(glp_ref, gln_ref, vq_ref, var_ref,
             vlp_ref, vln_ref, degp, degn,
             u1a, u1b, u1c, u1d, c1, u2, c2, u3, c3,
             y_ref, ps_ref):
    dpos = degp[...][:, 0:1]
    dneg = degn[...][:, 0:1]
    vdw = 4.0 * lax.rsqrt(jnp.maximum(dpos + dneg, 1.0))
    dwp = lax.rsqrt(jnp.maximum(dpos, 1.0))
    dwn = lax.rsqrt(jnp.maximum(dneg, 1.0))
    vq = vq_ref[...]
    vgrad = (glp_ref[...] * jax.nn.sigmoid(vq)
             - gln_ref[...] * jax.nn.sigmoid(-vq)) * vdw
    vlp = vlp_ref[...] * dwp
    vln = vln_ref[...] * dwn
    u = _mm(vgrad, u1a[...]) + _mm(var_ref[...], u1b[...]) \
        + _mm(vlp, u1c[...]) + _mm(vln, u1d[...]) + c1[...]
    u = jnp.maximum(u, 0.0)
    h2 = jnp.maximum(_mm(u, u2[...]) + c2[...], 0.0)
    y = _mm(h2, u3[...]) + c3[...]
    y_ref[...] = y
    ps_ref[...] = jnp.sum(y, axis=0, keepdims=True)[None]


def _k7_body(y_ref, ps_ref, var_ref, newv_ref):
    mean = jnp.sum(ps_ref[...][:, 0, :], axis=0, keepdims=True) * (1.0 / V)
    yc = y_ref[...] - mean
    var = jnp.mean(yc * yc, axis=1, keepdims=True)
    newv_ref[...] = yc * lax.rsqrt(var + 1e-6) * 0.25 + 0.1 * var_ref[...]


def _row_spec(b, d):
    return pl.BlockSpec((b, d), lambda i: (i, 0))


def _full_spec(shape):
    nd = len(shape)
    return pl.BlockSpec(shape, lambda i, _nd=nd: (0,) * _nd)


def kernel(lit_idx, clause_idx, clauses_graph, variables_graph, params):
    f32 = jnp.float32
    p = params

    qv_w1, qv_b1, qv_w2, qv_b2 = p["variables_query"]
    qc_w1, qc_b1, qc_w2, qc_b2 = p["clauses_query"]
    cm_w1, cm_b1, cm_w2, cm_b2 = p["clause_mlp"]
    ug_w1, ug_b1, ug_w2, ug_b2, ug_w3, ug_b3 = p["update_gate"]
    co_w1, co_b1, co_w2, co_b2 = p["clauses_output"]

    def r2(b):
        return b.reshape(1, -1)

    qv_w1v, qv_w1n = qv_w1[:FM], qv_w1[FM:]
    qc_w1v, qc_w1n = qc_w1[:FM], qc_w1[FM:]
    cm_w1a, cm_w1b, cm_w1c = cm_w1[:FM], cm_w1[FM:FM + QM], cm_w1[FM + QM:]
    ug_w1a, ug_w1b = ug_w1[:QM], ug_w1[QM:QM + FM]
    ug_w1c, ug_w1d = ug_w1[QM + FM:QM + 2 * FM], ug_w1[QM + 2 * FM:]

    pad = EP - E
    lit_g = jnp.pad(lit_idx, (0, pad)).reshape(EP // 128, 128)
    cla_g = jnp.pad(clause_idx, (0, pad)).reshape(EP // 128, 128)
    cla_pad = jnp.pad(clause_idx, (0, pad), constant_values=C)
    lit_pad = jnp.pad(lit_idx, (0, pad), constant_values=2 * V)
    zeros_e = jnp.zeros((EP // 128, 128), jnp.int32)

    def localize(idx, q):
        lo = q * QR
        return jnp.where((idx >= lo) & (idx < lo + QR), idx - lo,
                         QR + (idx & 7)).reshape(EP // 128, 128).astype(jnp.int32)

    sq_c = [localize(cla_pad, q) for q in range(4)]
    sq_l = [localize(lit_pad, q) for q in range(2)]
    z128 = jnp.zeros((64, 128), f32)
    ones_t = jnp.ones((128, 128), f32)

    deg = _segsum_deg(ones_t, zeros_e, sq_l, z128)

    variables = jnp.ones((V, FM), f32)
    clauses = jnp.ones((C, FM), f32)
    base_key = jax.random.key(1)

    k1 = pl.pallas_call(
        _k1_body,
        grid=(NBV,),
        in_specs=[_row_spec(BV, FM), _row_spec(BV, 4),
                  _full_spec((FM, QM)), _full_spec((4, QM)),
                  _full_spec((1, QM)), _full_spec((QM, QM)),
                  _full_spec((1, QM))],
        out_specs=_row_spec(BV, QM),
        out_shape=jax.ShapeDtypeStruct((V, QM), f32),
    )
    k1b = pl.pallas_call(
        _k1b_body,
        grid=(2, NBV),
        in_specs=[pl.BlockSpec((BV, QM), lambda h, i: (i, 0))],
        out_specs=pl.BlockSpec((BV, QM), lambda h, i: (h * NBV + i, 0)),
        out_shape=jax.ShapeDtypeStruct((2 * V, QM), f32),
    )
    k3 = pl.pallas_call(
        _k3_body,
        grid=(NBC,),
        in_specs=[_row_spec(BC, FM), _row_spec(BC, 4), _row_spec(BC, FM)]
        + [_full_spec((FM, QM)), _full_spec((4, QM)), _full_spec((1, QM)),
           _full_spec((QM, QM)), _full_spec((1, QM)),
           _full_spec((FM, 2 * FM)), _full_spec((QM, 2 * FM)),
           _full_spec((QM, 2 * FM)), _full_spec((1, 2 * FM)),
           _full_spec((2 * FM, FM + QM)), _full_spec((1, FM + QM))],
        out_specs=[_row_spec(BC, FM), _row_spec(BC, FM),
                   _row_spec(BC, FM),
                   pl.BlockSpec((1, 1, FM), lambda i: (i, 0, 0))],
        out_shape=[jax.ShapeDtypeStruct((C, FM), f32),
                   jax.ShapeDtypeStruct((C, FM), f32),
                   jax.ShapeDtypeStruct((C, FM), f32),
                   jax.ShapeDtypeStruct((NBC, 1, FM), f32)],
    )
    k4 = pl.pallas_call(
        _k4_body,
        grid=(NBC,),
        in_specs=[_row_spec(BC, FM), _full_spec((NBC, 1, FM)), _row_spec(BC, FM),
                  _full_spec((FM, FM)), _full_spec((1, FM)),
                  _full_spec((FM, 1)), _full_spec((1, 1))],
        out_specs=[_row_spec(BC, FM), _row_spec(BC, 1), _row_spec(BC, 1)],
        out_shape=[jax.ShapeDtypeStruct((C, FM), f32),
                   jax.ShapeDtypeStruct((C, 1), f32),
                   jax.ShapeDtypeStruct((C, 1), f32)],
    )
    neg_spec = pl.BlockSpec((BV, FM), lambda i: (i + NBV, 0))
    k6 = pl.pallas_call(
        _k6_body,
        grid=(NBV,),
        in_specs=[_row_spec(BV, FM), neg_spec,
                  _row_spec(BV, QM), _row_spec(BV, FM),
                  _row_spec(BV, FM), neg_spec,
                  _row_spec(BV, FM), neg_spec,
                  _full_spec((QM, 2 * FM)), _full_spec((FM, 2 * FM)),
                  _full_spec((FM, 2 * FM)), _full_spec((FM, 2 * FM)),
                  _full_spec((1, 2 * FM)),
                  _full_spec((2 * FM, 2 * FM)), _full_spec((1, 2 * FM)),
                  _full_spec((2 * FM, FM)), _full_spec((1, FM))],
        out_specs=[_row_spec(BV, FM),
                   pl.BlockSpec((1, 1, FM), lambda i: (i, 0, 0))],
        out_shape=[jax.ShapeDtypeStruct((V, FM), f32),
                   jax.ShapeDtypeStruct((NBV, 1, FM), f32)],
    )
    k7 = pl.pallas_call(
        _k7_body,
        grid=(NBV,),
        in_specs=[_row_spec(BV, FM), _full_spec((NBV, 1, FM)), _row_spec(BV, FM)],
        out_specs=_row_spec(BV, FM),
        out_shape=jax.ShapeDtypeStruct((V, FM), f32),
    )

    sig_rows = []
    sp_rows = []
    for step in range(ROUNDS):
        nv = jax.random.normal(jax.random.fold_in(base_key, 2 * step), (V, 4), f32)
        nc = jax.random.normal(jax.random.fold_in(base_key, 2 * step + 1), (C, 4), f32)

        vq = k1(variables, nv, qv_w1v, qv_w1n, r2(qv_b1), qv_w2, r2(qv_b2))
        lits = k1b(vq)
        csum = _segsum_c(lits, lit_g, sq_c, z128)
        vla, gcs, y_c, ps_c = k3(
            clauses, nc, csum,
            qc_w1v, qc_w1n, r2(qc_b1), qc_w2, r2(qc_b2),
            cm_w1a, cm_w1b, cm_w1c, r2(cm_b1), cm_w2, r2(cm_b2))
        clauses, sig_l, sp_l = k4(y_c, ps_c, clauses,
                                  co_w1, r2(co_b1), co_w2, r2(co_b2))
        sig_rows.append(sig_l)
        sp_rows.append(sp_l)
        gl = _segsum_l(gcs, cla_g, sq_l, z128)
        vl = _segsum_l(vla, cla_g, sq_l, z128)
        y_v, ps_v = k6(gl, gl, vq, variables, vl, vl, deg, deg,
                       ug_w1a, ug_w1b, ug_w1c, ug_w1d, r2(ug_b1),
                       ug_w2, r2(ug_b2), ug_w3, r2(ug_b3))
        variables = k7(y_v, ps_v, variables)

    sig = jnp.stack([jnp.squeeze(x, -1) for x in sig_rows], axis=0)
    sp = jnp.stack([jnp.squeeze(x, -1) for x in sp_rows], axis=0)
    return sig, sp


# speedup vs baseline: 3.0826x; 1.0823x over previous
import functools

import jax
import jax.numpy as jnp
from jax import lax
from jax.experimental import pallas as pl
from jax.experimental.pallas import tpu as pltpu
from jax.experimental.pallas import tpu_sc as plsc

V = 10000
C = 40000
E = 160000
FM = 128
QM = 128
ROUNDS = 4
EP = 163840
CHUNKS = EP // 16 // 128
BV = 1000
BC = 2000
NBV = V // BV
NBC = C // BC
S_LOSS = 1.0 / (C * QM)


def _mm(x, w):
    return lax.dot_general(x, w, (((1,), (0,)), ((), ())),
                           preferred_element_type=jnp.float32)


def _softplus(x):
    return jnp.maximum(x, 0.0) + jnp.log1p(jnp.exp(-jnp.abs(x)))


QR = 10000
RPT_HI = 632
RPT_TAIL = QR - 15 * RPT_HI


def _make_segsum(n_quarters, gather=True):
    n_per_core = n_quarters // 2
    mesh = plsc.VectorSubcoreMesh(core_axis_name="c", subcore_axis_name="s")
    n_out = n_quarters * QR

    ch2 = CHUNKS // 2

    @functools.partial(
        pl.kernel,
        mesh=mesh,
        out_type=jax.ShapeDtypeStruct((n_out, 128), jnp.float32),
        scratch_types=[
            pltpu.VMEM((ch2, 128), jnp.int32),
            pltpu.VMEM((ch2, 128), jnp.int32),
            pltpu.VMEM((128, 128), jnp.float32),
            pltpu.VMEM((128, 128), jnp.float32),
            pltpu.VMEM((32, 128), jnp.float32),
            pltpu.VMEM_SHARED((QR + 8, 128), jnp.float32),
            pltpu.SemaphoreType.DMA,
            pltpu.SemaphoreType.DMA,
        ],
    )
    def segsum(table, gidx, zsrc, *rest):
        sqs, (out, gv, sv, buf0, buf1, zbuf, acc, sem0, sem1) = (
            rest[:n_quarters], rest[n_quarters:])
        c = lax.axis_index("c")
        s = lax.axis_index("s")
        if not gather:
            pltpu.sync_copy(table, buf0)
        pltpu.sync_copy(zsrc, zbuf)
        bufs = ((buf0, sem0), (buf1, sem1))

        def run_pass(sq, q):
            base = s * RPT_HI

            def zero_rows(nrows):
                off = 0
                while off < nrows:
                    n = min(32, nrows - off)
                    pltpu.sync_copy(zbuf.at[pl.ds(0, n)],
                                    acc.at[pl.ds(base + off, n)])
                    off += n

            @pl.when(s < 15)
            def _():
                zero_rows(RPT_HI)

            @pl.when(s == 15)
            def _():
                zero_rows(RPT_TAIL)
                pltpu.sync_copy(zbuf.at[pl.ds(0, 8)], acc.at[pl.ds(QR, 8)])

            plsc.subcore_barrier()

            for h in range(2):
                cbase = s * CHUNKS + h * ch2
                pltpu.sync_copy(sq.at[pl.ds(cbase, ch2)], sv)
                if gather:
                    pltpu.sync_copy(gidx.at[pl.ds(cbase, ch2)], gv)
                    pltpu.async_copy(table.at[gv.at[0]], buf0, sem0)
                    pltpu.async_copy(table.at[gv.at[1]], buf1, sem1)

                    def pair(i, carry):
                        for b, (bufb, semb) in enumerate(bufs):
                            j = 2 * i + b
                            pltpu.make_async_copy(
                                table.at[gv.at[j]], bufb, semb).wait()
                            pltpu.sync_copy(bufb, acc.at[sv.at[j]], add=True)

                            @pl.when(j + 2 < ch2)
                            def _(bufb=bufb, semb=semb, j=j):
                                pltpu.async_copy(
                                    table.at[gv.at[j + 2]], bufb, semb)
                        return carry
                else:
                    def pair(i, carry):
                        for b in range(2):
                            j = 2 * i + b
                            pltpu.sync_copy(buf0, acc.at[sv.at[j]], add=True)
                        return carry

                lax.fori_loop(0, ch2 // 2, pair, 0)
            plsc.subcore_barrier()
            obase = q * QR + base

            @pl.when(s < 15)
            def _():
                pltpu.sync_copy(acc.at[pl.ds(base, RPT_HI)],
                                out.at[pl.ds(obase, RPT_HI)])

            @pl.when(s == 15)
            def _():
                pltpu.sync_copy(acc.at[pl.ds(base, RPT_TAIL)],
                                out.at[pl.ds(obase, RPT_TAIL)])

        for cid in range(2):
            @pl.when(c == cid)
            def _(cid=cid):
                for p in range(n_per_core):
                    run_pass(sqs[cid * n_per_core + p], cid * n_per_core + p)

    def call(table, gidx, sqs, zsrc):
        return segsum(table, gidx, zsrc, *sqs)

    return call


_segsum_c = _make_segsum(4)
_segsum_l = _make_segsum(2)
_segsum_deg = _make_segsum(2, gather=False)


def _k1_body(var_ref, nv_ref, w1v, w1n, b1, w2, b2, vq_ref):
    h = _mm(var_ref[...], w1v[...]) + _mm(nv_ref[...], w1n[...]) + b1[...]
    h = jnp.maximum(h, 0.0)
    vq_ref[...] = _mm(h, w2[...]) + b2[...]


def _k1b_body(vq_ref, lits_ref):
    sign = jnp.where(pl.program_id(0) == 0, 1.0, -1.0)
    lits_ref[...] = _softplus(vq_ref[...] * sign)


def _k3_body(cl_ref, nc_ref, cs_ref,
             qw1v, qw1n, qb1, qw2, qb2,
             cw1a, cw1b, cw1c, cb1, cw2, cb2,
             vla_ref, g_ref, y_ref, ps_ref):
    clauses = cl_ref[...]
    h = _mm(clauses, qw1v[...]) + _mm(nc_ref[...], qw1n[...]) + qb1[...]
    h = jnp.maximum(h, 0.0)
    cq = _mm(h, qw2[...]) + qb2[...]
    sig = jax.nn.sigmoid(cq)
    cl = jnp.exp(-cs_ref[...]) * sig
    cgrad = S_LOSS * cl * (1.0 - sig)
    g_ref[...] = -S_LOSS * cl
    u = _mm(clauses, cw1a[...]) + _mm(4.0 * cl, cw1b[...]) \
        + _mm(cgrad, cw1c[...]) + cb1[...]
    u = jnp.maximum(u, 0.0)
    d = _mm(u, cw2[...]) + cb2[...]
    vla_ref[...] = d[:, 0:128]
    y = d[:, 128:256]
    y_ref[...] = y
    ps_ref[...] = jnp.sum(y, axis=0, keepdims=True)[None]


def _k4_body(y_ref, ps_ref, cl_ref, ow1, ob1, ow2, ob2,
             newc_ref, sig_ref, sp_ref):
    mean = jnp.sum(ps_ref[...][:, 0, :], axis=0, keepdims=True) * (1.0 / C)
    yc = y_ref[...] - mean
    var = jnp.mean(yc * yc, axis=1, keepdims=True)
    newc = yc * lax.rsqrt(var + 1e-6) * 0.25 + 0.1 * cl_ref[...]
    newc_ref[...] = newc
    hh = jnp.maximum(_mm(newc, ow1[...]) + ob1[...], 0.0)
    logit = _mm(hh, ow2[...]) + ob2[...]
    sig_ref[...] = jax.nn.sigmoid(logit)
    sp_ref[...] = _softplus(logit)


def _k6_body(glp_ref, gln_ref, vq_ref, var_ref,
             vlp_ref, vln_ref, degp, degn,
             u1a, u1b, u1c, u1d, c1, u2, c2, u3, c3,
             y_ref, ps_ref):
    dpos = degp[...][:, 0:1]
    dneg = degn[...][:, 0:1]
    vdw = 4.0 * lax.rsqrt(jnp.maximum(dpos + dneg, 1.0))
    dwp = lax.rsqrt(jnp.maximum(dpos, 1.0))
    dwn = lax.rsqrt(jnp.maximum(dneg, 1.0))
    vq = vq_ref[...]
    vgrad = (glp_ref[...] * jax.nn.sigmoid(vq)
             - gln_ref[...] * jax.nn.sigmoid(-vq)) * vdw
    vlp = vlp_ref[...] * dwp
    vln = vln_ref[...] * dwn
    u = _mm(vgrad, u1a[...]) + _mm(var_ref[...], u1b[...]) \
        + _mm(vlp, u1c[...]) + _mm(vln, u1d[...]) + c1[...]
    u = jnp.maximum(u, 0.0)
    h2 = jnp.maximum(_mm(u, u2[...]) + c2[...], 0.0)
    y = _mm(h2, u3[...]) + c3[...]
    y_ref[...] = y
    ps_ref[...] = jnp.sum(y, axis=0, keepdims=True)[None]


def _k7_body(y_ref, ps_ref, var_ref, newv_ref):
    mean = jnp.sum(ps_ref[...][:, 0, :], axis=0, keepdims=True) * (1.0 / V)
    yc = y_ref[...] - mean
    var = jnp.mean(yc * yc, axis=1, keepdims=True)
    newv_ref[...] = yc * lax.rsqrt(var + 1e-6) * 0.25 + 0.1 * var_ref[...]


def _row_spec(b, d):
    return pl.BlockSpec((b, d), lambda i: (i, 0))


def _full_spec(shape):
    nd = len(shape)
    return pl.BlockSpec(shape, lambda i, _nd=nd: (0,) * _nd)


def kernel(lit_idx, clause_idx, clauses_graph, variables_graph, params):
    f32 = jnp.float32
    p = params

    qv_w1, qv_b1, qv_w2, qv_b2 = p["variables_query"]
    qc_w1, qc_b1, qc_w2, qc_b2 = p["clauses_query"]
    cm_w1, cm_b1, cm_w2, cm_b2 = p["clause_mlp"]
    ug_w1, ug_b1, ug_w2, ug_b2, ug_w3, ug_b3 = p["update_gate"]
    co_w1, co_b1, co_w2, co_b2 = p["clauses_output"]

    def r2(b):
        return b.reshape(1, -1)

    qv_w1v, qv_w1n = qv_w1[:FM], qv_w1[FM:]
    qc_w1v, qc_w1n = qc_w1[:FM], qc_w1[FM:]
    cm_w1a, cm_w1b, cm_w1c = cm_w1[:FM], cm_w1[FM:FM + QM], cm_w1[FM + QM:]
    ug_w1a, ug_w1b = ug_w1[:QM], ug_w1[QM:QM + FM]
    ug_w1c, ug_w1d = ug_w1[QM + FM:QM + 2 * FM], ug_w1[QM + 2 * FM:]

    pad = EP - E
    lit_g = jnp.pad(lit_idx, (0, pad)).reshape(EP // 128, 128)
    cla_g = jnp.pad(clause_idx, (0, pad)).reshape(EP // 128, 128)
    cla_pad = jnp.pad(clause_idx, (0, pad), constant_values=C)
    lit_pad = jnp.pad(lit_idx, (0, pad), constant_values=2 * V)
    zeros_e = jnp.zeros((EP // 128, 128), jnp.int32)

    def localize(idx, q):
        lo = q * QR
        return jnp.where((idx >= lo) & (idx < lo + QR), idx - lo,
                         QR + (idx & 7)).reshape(EP // 128, 128).astype(jnp.int32)

    sq_c = [localize(cla_pad, q) for q in range(4)]
    sq_l = [localize(lit_pad, q) for q in range(2)]
    z128 = jnp.zeros((32, 128), f32)
    ones_t = jnp.ones((128, 128), f32)

    deg = _segsum_deg(ones_t, zeros_e, sq_l, z128)

    variables = jnp.ones((V, FM), f32)
    clauses = jnp.ones((C, FM), f32)
    base_key = jax.random.key(1)

    k1 = pl.pallas_call(
        _k1_body,
        grid=(NBV,),
        in_specs=[_row_spec(BV, FM), _row_spec(BV, 4),
                  _full_spec((FM, QM)), _full_spec((4, QM)),
                  _full_spec((1, QM)), _full_spec((QM, QM)),
                  _full_spec((1, QM))],
        out_specs=_row_spec(BV, QM),
        out_shape=jax.ShapeDtypeStruct((V, QM), f32),
    )
    k1b = pl.pallas_call(
        _k1b_body,
        grid=(2, NBV),
        in_specs=[pl.BlockSpec((BV, QM), lambda h, i: (i, 0))],
        out_specs=pl.BlockSpec((BV, QM), lambda h, i: (h * NBV + i, 0)),
        out_shape=jax.ShapeDtypeStruct((2 * V, QM), f32),
    )
    k3 = pl.pallas_call(
        _k3_body,
        grid=(NBC,),
        in_specs=[_row_spec(BC, FM), _row_spec(BC, 4), _row_spec(BC, FM)]
        + [_full_spec((FM, QM)), _full_spec((4, QM)), _full_spec((1, QM)),
           _full_spec((QM, QM)), _full_spec((1, QM)),
           _full_spec((FM, 2 * FM)), _full_spec((QM, 2 * FM)),
           _full_spec((QM, 2 * FM)), _full_spec((1, 2 * FM)),
           _full_spec((2 * FM, FM + QM)), _full_spec((1, FM + QM))],
        out_specs=[_row_spec(BC, FM), _row_spec(BC, FM),
                   _row_spec(BC, FM),
                   pl.BlockSpec((1, 1, FM), lambda i: (i, 0, 0))],
        out_shape=[jax.ShapeDtypeStruct((C, FM), f32),
                   jax.ShapeDtypeStruct((C, FM), f32),
                   jax.ShapeDtypeStruct((C, FM), f32),
                   jax.ShapeDtypeStruct((NBC, 1, FM), f32)],
    )
    k4 = pl.pallas_call(
        _k4_body,
        grid=(NBC,),
        in_specs=[_row_spec(BC, FM), _full_spec((NBC, 1, FM)), _row_spec(BC, FM),
                  _full_spec((FM, FM)), _full_spec((1, FM)),
                  _full_spec((FM, 1)), _full_spec((1, 1))],
        out_specs=[_row_spec(BC, FM), _row_spec(BC, 1), _row_spec(BC, 1)],
        out_shape=[jax.ShapeDtypeStruct((C, FM), f32),
                   jax.ShapeDtypeStruct((C, 1), f32),
                   jax.ShapeDtypeStruct((C, 1), f32)],
    )
    neg_spec = pl.BlockSpec((BV, FM), lambda i: (i + NBV, 0))
    k6 = pl.pallas_call(
        _k6_body,
        grid=(NBV,),
        in_specs=[_row_spec(BV, FM), neg_spec,
                  _row_spec(BV, QM), _row_spec(BV, FM),
                  _row_spec(BV, FM), neg_spec,
                  _row_spec(BV, FM), neg_spec,
                  _full_spec((QM, 2 * FM)), _full_spec((FM, 2 * FM)),
                  _full_spec((FM, 2 * FM)), _full_spec((FM, 2 * FM)),
                  _full_spec((1, 2 * FM)),
                  _full_spec((2 * FM, 2 * FM)), _full_spec((1, 2 * FM)),
                  _full_spec((2 * FM, FM)), _full_spec((1, FM))],
        out_specs=[_row_spec(BV, FM),
                   pl.BlockSpec((1, 1, FM), lambda i: (i, 0, 0))],
        out_shape=[jax.ShapeDtypeStruct((V, FM), f32),
                   jax.ShapeDtypeStruct((NBV, 1, FM), f32)],
    )
    k7 = pl.pallas_call(
        _k7_body,
        grid=(NBV,),
        in_specs=[_row_spec(BV, FM), _full_spec((NBV, 1, FM)), _row_spec(BV, FM)],
        out_specs=_row_spec(BV, FM),
        out_shape=jax.ShapeDtypeStruct((V, FM), f32),
    )

    sig_rows = []
    sp_rows = []
    for step in range(ROUNDS):
        nv = jax.random.normal(jax.random.fold_in(base_key, 2 * step), (V, 4), f32)
        nc = jax.random.normal(jax.random.fold_in(base_key, 2 * step + 1), (C, 4), f32)

        vq = k1(variables, nv, qv_w1v, qv_w1n, r2(qv_b1), qv_w2, r2(qv_b2))
        lits = k1b(vq)
        csum = _segsum_c(lits, lit_g, sq_c, z128)
        vla, gcs, y_c, ps_c = k3(
            clauses, nc, csum,
            qc_w1v, qc_w1n, r2(qc_b1), qc_w2, r2(qc_b2),
            cm_w1a, cm_w1b, cm_w1c, r2(cm_b1), cm_w2, r2(cm_b2))
        clauses, sig_l, sp_l = k4(y_c, ps_c, clauses,
                                  co_w1, r2(co_b1), co_w2, r2(co_b2))
        sig_rows.append(sig_l)
        sp_rows.append(sp_l)
        gl = _segsum_l(gcs, cla_g, sq_l, z128)
        vl = _segsum_l(vla, cla_g, sq_l, z128)
        y_v, ps_v = k6(gl, gl, vq, variables, vl, vl, deg, deg,
                       ug_w1a, ug_w1b, ug_w1c, ug_w1d, r2(ug_b1),
                       ug_w2, r2(ug_b2), ug_w3, r2(ug_b3))
        variables = k7(y_v, ps_v, variables)

    sig = jnp.stack([jnp.squeeze(x, -1) for x in sig_rows], axis=0)
    sp = jnp.stack([jnp.squeeze(x, -1) for x in sp_rows], axis=0)
    return sig, sp
